# baseline (speedup 1.0000x reference)
import functools

import jax
import jax.numpy as jnp
from jax import lax
from jax.experimental import pallas as pl
from jax.experimental.pallas import tpu as pltpu
from jax.experimental.pallas import tpu_sc as plsc

_D = 32
_NC = 2
_NS = 16
_NW = _NC * _NS
_S = 16384
_W = 50
_B = _S * _W
_SPW = _S // _NW
_SPC = 16
_CHUNK = _SPC * _W
_NCH = _SPW // _SPC
_NBUF = 2
_L = 16


@functools.partial(
    pl.kernel,
    out_type=jax.ShapeDtypeStruct((_W, _D, _S), jnp.float32),
    mesh=plsc.VectorSubcoreMesh(core_axis_name="c", subcore_axis_name="s"),
    scratch_types=[
        [pltpu.VMEM((_CHUNK,), jnp.int32)] * _NBUF,
        [pltpu.VMEM((_CHUNK, _D), jnp.float32)] * _NBUF,
        [pltpu.VMEM((_W, _D, _SPC), jnp.float32)] * _NBUF,
        [pltpu.SemaphoreType.DMA] * _NBUF,
        [pltpu.SemaphoreType.DMA] * _NBUF,
        [pltpu.SemaphoreType.DMA] * _NBUF,
    ],
    compiler_params=pltpu.CompilerParams(
        use_tc_tiling_on_sc=False, needs_layout_passes=False
    ),
)
def _gather_body(ids_hbm, table_hbm, out_hbm, idxb, rows, trows, isem, gsem, wsem):
    wid = lax.axis_index("s") * _NC + lax.axis_index("c")
    sbase = wid * _SPW
    wbase = sbase * _W

    def istart(c, b):
        pltpu.async_copy(
            ids_hbm.at[pl.ds(wbase + c * _CHUNK, _CHUNK)], idxb[b], isem[b]
        )

    def iwait(c, b):
        pltpu.make_async_copy(
            ids_hbm.at[pl.ds(wbase + c * _CHUNK, _CHUNK)], idxb[b], isem[b]
        ).wait()

    def gstart(c, b):
        pltpu.async_copy(table_hbm.at[idxb[b]], rows[b], gsem[b])

    def gwait(c, b):
        pltpu.make_async_copy(table_hbm.at[idxb[b]], rows[b], gsem[b]).wait()

    def wstart(c, b):
        pltpu.async_copy(
            trows[b],
            out_hbm.at[:, :, pl.ds(sbase + c * _SPC, _SPC)],
            wsem[b],
        )

    def wwait(c, b):
        pltpu.make_async_copy(
            trows[b],
            out_hbm.at[:, :, pl.ds(sbase + c * _SPC, _SPC)],
            wsem[b],
        ).wait()

    lane = lax.iota(jnp.int32, _L)
    srow = lane * _W
    dcols = [jnp.full((_L,), d, jnp.int32) for d in range(_D)]

    def transpose_chunk(b):
        def per_w(w, carry):
            ridx = srow + w
            for d in range(_D):
                vals = plsc.load_gather(rows[b], [ridx, dcols[d]])
                trows[b][w, d] = vals
            return carry

        lax.fori_loop(0, _W, per_w, 0)

    istart(0, 0)
    iwait(0, 0)
    gstart(0, 0)
    istart(1, 1)

    def group(i, carry):
        for b in range(_NBUF):
            c = i * _NBUF + b
            nb = (b + _NBUF - 1) % _NBUF
            gwait(c, b)

            n = c + _NBUF - 1

            @pl.when(n < _NCH)
            def _():
                iwait(n, nb)
                gstart(n, nb)

            @pl.when(c >= _NBUF)
            def _():
                wwait(c - _NBUF, b)

            transpose_chunk(b)
            wstart(c, b)

            m = c + _NBUF

            @pl.when(m < _NCH)
            def _():
                istart(m, b)

        return carry

    lax.fori_loop(0, _NCH // _NBUF, group, 0)

    for k in range(_NBUF):
        c = _NCH - _NBUF + k
        wwait(c, c % _NBUF)


@jax.jit
def _run(ids, table):
    out = _gather_body(ids, table)
    return jnp.transpose(out, (2, 0, 1))


def kernel(token_ids, weights):
    ids = token_ids.reshape(-1).astype(jnp.int32)
    return _run(ids, weights)


# speedup vs baseline: 1.4886x; 1.4886x over previous
import functools

import jax
import jax.numpy as jnp
from jax import lax
from jax.experimental import pallas as pl
from jax.experimental.pallas import tpu as pltpu
from jax.experimental.pallas import tpu_sc as plsc

_D = 32
_NC = 2
_NS = 16
_NW = _NC * _NS
_S = 16384
_W = 50
_B = _S * _W
_SPW = _S // _NW
_SPC = 16
_CHUNK = _SPC * _W
_NCH = _SPW // _SPC
_NBUF = 2
_L = 16


@functools.partial(
    pl.kernel,
    out_type=jax.ShapeDtypeStruct((_W, _D, _S), jnp.float32),
    mesh=plsc.VectorSubcoreMesh(core_axis_name="c", subcore_axis_name="s"),
    scratch_types=[
        [pltpu.VMEM((_CHUNK,), jnp.int32)] * _NBUF,
        [pltpu.VMEM((_CHUNK, _D), jnp.float32)] * _NBUF,
        [pltpu.VMEM((_W, _D, _SPC), jnp.float32)] * _NBUF,
        [pltpu.SemaphoreType.DMA] * _NBUF,
        [pltpu.SemaphoreType.DMA] * _NBUF,
        [pltpu.SemaphoreType.DMA] * _NBUF,
    ],
    compiler_params=pltpu.CompilerParams(
        use_tc_tiling_on_sc=False, needs_layout_passes=False
    ),
)
def _gather_body(ids_hbm, table_hbm, out_hbm, idxb, rows, trows, isem, gsem, wsem):
    wid = lax.axis_index("s") * _NC + lax.axis_index("c")
    sbase = wid * _SPW
    wbase = sbase * _W

    def istart(c, b):
        pltpu.async_copy(
            ids_hbm.at[pl.ds(wbase + c * _CHUNK, _CHUNK)], idxb[b], isem[b]
        )

    def iwait(c, b):
        pltpu.make_async_copy(
            ids_hbm.at[pl.ds(wbase + c * _CHUNK, _CHUNK)], idxb[b], isem[b]
        ).wait()

    def gstart(c, b):
        pltpu.async_copy(table_hbm.at[idxb[b]], rows[b], gsem[b])

    def gwait(c, b):
        pltpu.make_async_copy(table_hbm.at[idxb[b]], rows[b], gsem[b]).wait()

    def wstart(c, b):
        pltpu.async_copy(
            trows[b],
            out_hbm.at[:, :, pl.ds(sbase + c * _SPC, _SPC)],
            wsem[b],
        )

    def wwait(c, b):
        pltpu.make_async_copy(
            trows[b],
            out_hbm.at[:, :, pl.ds(sbase + c * _SPC, _SPC)],
            wsem[b],
        ).wait()

    lane = lax.iota(jnp.int32, _L)
    srow = lane * _W
    dvecs = [
        j * _L + (lane + i) % _L
        for j in range(_D // _L)
        for i in range(_L)
    ]

    def transpose_chunk(b):
        def per_w(w, carry):
            ridx = srow + w
            wvec = jnp.full((_L,), 0, jnp.int32) + w
            for dvec in dvecs:
                vals = plsc.load_gather(rows[b], [ridx, dvec])
                plsc.store_scatter(trows[b], [wvec, dvec, lane], vals)
            return carry

        lax.fori_loop(0, _W, per_w, 0)

    istart(0, 0)
    iwait(0, 0)
    gstart(0, 0)
    istart(1, 1)

    def group(i, carry):
        for b in range(_NBUF):
            c = i * _NBUF + b
            nb = (b + _NBUF - 1) % _NBUF
            gwait(c, b)

            n = c + _NBUF - 1

            @pl.when(n < _NCH)
            def _():
                iwait(n, nb)
                gstart(n, nb)

            @pl.when(c >= _NBUF)
            def _():
                wwait(c - _NBUF, b)

            transpose_chunk(b)
            wstart(c, b)

            m = c + _NBUF

            @pl.when(m < _NCH)
            def _():
                istart(m, b)

        return carry

    lax.fori_loop(0, _NCH // _NBUF, group, 0)

    for k in range(_NBUF):
        c = _NCH - _NBUF + k
        wwait(c, c % _NBUF)


@jax.jit
def _run(ids, table):
    out = _gather_body(ids, table)
    return jnp.transpose(out, (2, 0, 1))


def kernel(token_ids, weights):
    ids = token_ids.reshape(-1).astype(jnp.int32)
    return _run(ids, weights)


# speedup vs baseline: 1.5049x; 1.0109x over previous
import functools

import jax
import jax.numpy as jnp
from jax import lax
from jax.experimental import pallas as pl
from jax.experimental.pallas import tpu as pltpu
from jax.experimental.pallas import tpu_sc as plsc

_D = 32
_NC = 2
_NS = 16
_NW = _NC * _NS
_S = 16384
_W = 50
_B = _S * _W
_SPW = _S // _NW
_SPC = 16
_CHUNK = _SPC * _W
_NCH = _SPW // _SPC
_NBUF = 2
_L = 16


@functools.partial(
    pl.kernel,
    out_type=jax.ShapeDtypeStruct((_W, _D, _S), jnp.float32),
    mesh=plsc.VectorSubcoreMesh(core_axis_name="c", subcore_axis_name="s"),
    scratch_types=[
        [pltpu.VMEM((_CHUNK,), jnp.int32)] * _NBUF,
        [pltpu.VMEM((_CHUNK, _D), jnp.float32)] * _NBUF,
        [pltpu.VMEM((_W, _D, _SPC), jnp.float32)] * _NBUF,
        [pltpu.SemaphoreType.DMA] * _NBUF,
        [pltpu.SemaphoreType.DMA] * _NBUF,
        [pltpu.SemaphoreType.DMA] * _NBUF,
    ],
    compiler_params=pltpu.CompilerParams(
        use_tc_tiling_on_sc=False, needs_layout_passes=False
    ),
)
def _gather_body(ids_hbm, table_hbm, out_hbm, idxb, rows, trows, isem, gsem, wsem):
    wid = lax.axis_index("s") * _NC + lax.axis_index("c")
    sbase = wid * _SPW
    wbase = sbase * _W

    def istart(c, b):
        pltpu.async_copy(
            ids_hbm.at[pl.ds(wbase + c * _CHUNK, _CHUNK)], idxb[b], isem[b]
        )

    def iwait(c, b):
        pltpu.make_async_copy(
            ids_hbm.at[pl.ds(wbase + c * _CHUNK, _CHUNK)], idxb[b], isem[b]
        ).wait()

    def gstart(c, b):
        pltpu.async_copy(table_hbm.at[idxb[b]], rows[b], gsem[b])

    def gwait(c, b):
        pltpu.make_async_copy(table_hbm.at[idxb[b]], rows[b], gsem[b]).wait()

    def wstart(c, b):
        pltpu.async_copy(
            trows[b],
            out_hbm.at[:, :, pl.ds(sbase + c * _SPC, _SPC)],
            wsem[b],
        )

    def wwait(c, b):
        pltpu.make_async_copy(
            trows[b],
            out_hbm.at[:, :, pl.ds(sbase + c * _SPC, _SPC)],
            wsem[b],
        ).wait()

    lane = lax.iota(jnp.int32, _L)
    srow = lane * _W
    dvecs = [
        j * _L + (lane + i) % _L
        for j in range(_D // _L)
        for i in range(_L)
    ]

    def transpose_chunk(b):
        def per_w(i, carry):
            w0 = i * 2
            for u in range(2):
                w = w0 + u
                ridx = srow + w
                wvec = jnp.full((_L,), 0, jnp.int32) + w
                for dvec in dvecs:
                    vals = plsc.load_gather(rows[b], [ridx, dvec])
                    plsc.store_scatter(trows[b], [wvec, dvec, lane], vals)
            return carry

        lax.fori_loop(0, _W // 2, per_w, 0)

    istart(0, 0)
    iwait(0, 0)
    gstart(0, 0)
    istart(1, 1)

    def group(i, carry):
        for b in range(_NBUF):
            c = i * _NBUF + b
            nb = (b + _NBUF - 1) % _NBUF
            gwait(c, b)

            n = c + _NBUF - 1

            @pl.when(n < _NCH)
            def _():
                iwait(n, nb)
                gstart(n, nb)

            @pl.when(c >= _NBUF)
            def _():
                wwait(c - _NBUF, b)

            transpose_chunk(b)
            wstart(c, b)

            m = c + _NBUF

            @pl.when(m < _NCH)
            def _():
                istart(m, b)

        return carry

    lax.fori_loop(0, _NCH // _NBUF, group, 0)

    for k in range(_NBUF):
        c = _NCH - _NBUF + k
        wwait(c, c % _NBUF)


@jax.jit
def _run(ids, table):
    out = _gather_body(ids, table)
    return jnp.transpose(out, (2, 0, 1))


def kernel(token_ids, weights):
    ids = token_ids.reshape(-1).astype(jnp.int32)
    return _run(ids, weights)
